# baseline (speedup 1.0000x reference)
import jax
import jax.numpy as jnp
from jax.experimental import pallas as pl

EPS = 1e-06
NUM_GROUPS = 4
BLOCK_T = 1024


def _glnorm_kernel(x_ref, tt_ref, g_ref, b_ref, o_ref):
    x = x_ref[...]
    tt = tt_ref[...]
    d = x.shape[1]
    hi = jax.lax.Precision.HIGHEST
    ones = jnp.ones((d, 1), jnp.float32)
    s1 = jax.lax.dot(x, ones, precision=hi)
    s2 = jax.lax.dot(x * x, ones, precision=hi)
    mean = s1 * (1.0 / d)
    var = s2 * (1.0 / d) - mean * mean
    inv = jax.lax.rsqrt(var + EPS)
    onehot = (tt == jnp.arange(NUM_GROUPS)[None, :]).astype(jnp.float32)
    gg = jax.lax.dot(onehot, g_ref[...], precision=hi)
    bb = jax.lax.dot(onehot, b_ref[...], precision=hi)
    a = gg * inv
    o_ref[...] = x * a + (bb - mean * a)


def kernel(x, token_types, gamma, beta):
    B, S, D = x.shape
    n_tok = B * S
    x2 = x.reshape(n_tok, D)
    tt2 = token_types.reshape(n_tok, 1).astype(jnp.int32)
    grid = (n_tok // BLOCK_T,)
    out = pl.pallas_call(
        _glnorm_kernel,
        grid=grid,
        in_specs=[
            pl.BlockSpec((BLOCK_T, D), lambda i: (i, 0)),
            pl.BlockSpec((BLOCK_T, 1), lambda i: (i, 0)),
            pl.BlockSpec((NUM_GROUPS, D), lambda i: (0, 0)),
            pl.BlockSpec((NUM_GROUPS, D), lambda i: (0, 0)),
        ],
        out_specs=pl.BlockSpec((BLOCK_T, D), lambda i: (i, 0)),
        out_shape=jax.ShapeDtypeStruct((n_tok, D), x.dtype),
    )(x2, tt2, gamma, beta)
    return out.reshape(B, S, D)


# speedup vs baseline: 1.8856x; 1.8856x over previous
import jax
import jax.numpy as jnp
from jax.experimental import pallas as pl

EPS = 1e-06
NUM_GROUPS = 4
BLOCK_T = 1024


def _glnorm_kernel(x_ref, tt_ref, g_ref, b_ref, o_ref):
    x = x_ref[...]
    tt = tt_ref[...]
    d = x.shape[1]
    hi = jax.lax.Precision.HIGHEST
    mean = jnp.mean(x, axis=1, keepdims=True)
    xc = x - mean
    var = jnp.mean(xc * xc, axis=1, keepdims=True)
    inv = jax.lax.rsqrt(var + EPS)
    onehot = (tt == jnp.arange(NUM_GROUPS)[None, :]).astype(jnp.float32)
    gg = jax.lax.dot(onehot, g_ref[...], precision=hi)
    bb = jax.lax.dot(onehot, b_ref[...], precision=hi)
    o_ref[...] = xc * (inv * gg) + bb


def kernel(x, token_types, gamma, beta):
    B, S, D = x.shape
    n_tok = B * S
    x2 = x.reshape(n_tok, D)
    tt2 = token_types.reshape(n_tok, 1).astype(jnp.int32)
    grid = (n_tok // BLOCK_T,)
    out = pl.pallas_call(
        _glnorm_kernel,
        grid=grid,
        in_specs=[
            pl.BlockSpec((BLOCK_T, D), lambda i: (i, 0)),
            pl.BlockSpec((BLOCK_T, 1), lambda i: (i, 0)),
            pl.BlockSpec((NUM_GROUPS, D), lambda i: (0, 0)),
            pl.BlockSpec((NUM_GROUPS, D), lambda i: (0, 0)),
        ],
        out_specs=pl.BlockSpec((BLOCK_T, D), lambda i: (i, 0)),
        out_shape=jax.ShapeDtypeStruct((n_tok, D), x.dtype),
    )(x2, tt2, gamma, beta)
    return out.reshape(B, S, D)


# speedup vs baseline: 3.9720x; 2.1064x over previous
import jax
import jax.numpy as jnp
from jax.experimental import pallas as pl

EPS = 1e-06
NUM_GROUPS = 4
BLOCK_T = 1024


def _glnorm_kernel(x_ref, tt_ref, g_ref, b_ref, o_ref):
    x = x_ref[...]
    tt = tt_ref[...]
    d = x.shape[1]
    mean = jnp.mean(x, axis=1, keepdims=True)
    xc = x - mean
    var = jnp.mean(xc * xc, axis=1, keepdims=True)
    inv = jax.lax.rsqrt(var + EPS)
    onehot = (tt == jnp.arange(NUM_GROUPS)[None, :]).astype(jnp.float32)
    gg = jax.lax.dot(onehot, g_ref[...])
    bb = jax.lax.dot(onehot, b_ref[...])
    o_ref[...] = xc * (inv * gg) + bb


def kernel(x, token_types, gamma, beta):
    B, S, D = x.shape
    n_tok = B * S
    x2 = x.reshape(n_tok, D)
    tt2 = token_types.reshape(n_tok, 1).astype(jnp.int32)
    grid = (n_tok // BLOCK_T,)
    out = pl.pallas_call(
        _glnorm_kernel,
        grid=grid,
        in_specs=[
            pl.BlockSpec((BLOCK_T, D), lambda i: (i, 0)),
            pl.BlockSpec((BLOCK_T, 1), lambda i: (i, 0)),
            pl.BlockSpec((NUM_GROUPS, D), lambda i: (0, 0)),
            pl.BlockSpec((NUM_GROUPS, D), lambda i: (0, 0)),
        ],
        out_specs=pl.BlockSpec((BLOCK_T, D), lambda i: (i, 0)),
        out_shape=jax.ShapeDtypeStruct((n_tok, D), x.dtype),
    )(x2, tt2, gamma, beta)
    return out.reshape(B, S, D)
